# baseline (speedup 1.0000x reference)
import functools

import jax
import jax.numpy as jnp
from jax import lax
from jax.experimental import pallas as pl
from jax.experimental.pallas import tpu as pltpu
from jax.experimental.pallas import tpu_sc as plsc

_NC = 2
_NS = 16
_NW = _NC * _NS
_NBUF = 2


@functools.partial(jax.jit, static_argnames=("d_model", "chunk"))
def _sc_embed(idx, W_E, d_model, chunk):
    n_chunks = idx.shape[1]
    total = _NW * n_chunks * chunk
    n_rounds = n_chunks // _NBUF
    mesh = plsc.VectorSubcoreMesh(core_axis_name="c", subcore_axis_name="s")

    @functools.partial(
        pl.kernel,
        out_type=jax.ShapeDtypeStruct((total, d_model), jnp.float32),
        mesh=mesh,
        scratch_types=[
            pltpu.VMEM((n_chunks, chunk), jnp.int32),
            pltpu.VMEM((_NBUF, chunk, d_model), jnp.float32),
            [pltpu.SemaphoreType.DMA] * _NBUF,
            [pltpu.SemaphoreType.DMA] * _NBUF,
        ],
    )
    def k(idx_hbm, table_hbm, out_hbm, idx_v, bufs, gsems, ssems):
        wid = lax.axis_index("s") * _NC + lax.axis_index("c")
        base = wid * n_chunks * chunk
        pltpu.sync_copy(idx_hbm.at[wid], idx_v)

        def gather(g, b):
            return pltpu.make_async_copy(
                table_hbm.at[idx_v.at[g]], bufs.at[b], gsems[b]
            )

        def store(g, b):
            return pltpu.make_async_copy(
                bufs.at[b], out_hbm.at[pl.ds(base + g * chunk, chunk)], ssems[b]
            )

        for b in range(_NBUF):
            gather(b, b).start()

        def body(i, carry):
            g0 = i * _NBUF
            for b in range(_NBUF):
                gather(g0 + b, b).wait()
                store(g0 + b, b).start()
            for b in range(_NBUF):
                gnext = g0 + _NBUF + b

                @pl.when(gnext < n_chunks)
                def _():
                    store(g0 + b, b).wait()
                    gather(gnext, b).start()

            return carry

        lax.fori_loop(0, n_rounds, body, 0, unroll=False)

        for b in range(_NBUF):
            store(n_chunks - _NBUF + b, b).wait()

    return k(idx, W_E)


def kernel(tokens, W_E):
    B, T = tokens.shape
    V, D = W_E.shape
    total = B * T
    chunk = 32
    n_chunks = total // (_NW * chunk)
    idx = tokens.reshape(_NW, n_chunks, chunk).astype(jnp.int32)
    out = _sc_embed(idx, W_E, D, chunk)
    return out.reshape(B, T, D)


# speedup vs baseline: 1.0595x; 1.0595x over previous
import functools

import jax
import jax.numpy as jnp
from jax import lax
from jax.experimental import pallas as pl
from jax.experimental.pallas import tpu as pltpu
from jax.experimental.pallas import tpu_sc as plsc

_NC = 2
_NS = 16
_NW = _NC * _NS
_CHUNK = 56


@functools.partial(jax.jit, static_argnames=("d_model",))
def _sc_embed(idx, W_E, d_model):
    n_per = idx.shape[1]
    total = _NW * n_per
    sizes = [_CHUNK] * (n_per // _CHUNK)
    if n_per % _CHUNK:
        sizes.append(n_per % _CHUNK)
    offs = [sum(sizes[:j]) for j in range(len(sizes))]
    n = len(sizes)
    mesh = plsc.VectorSubcoreMesh(core_axis_name="c", subcore_axis_name="s")

    @functools.partial(
        pl.kernel,
        out_type=jax.ShapeDtypeStruct((total, d_model), jnp.float32),
        mesh=mesh,
        scratch_types=[
            pltpu.VMEM((n_per,), jnp.int32),
            pltpu.VMEM((_CHUNK, d_model), jnp.float32),
            pltpu.VMEM((_CHUNK, d_model), jnp.float32),
            pltpu.SemaphoreType.DMA,
            pltpu.SemaphoreType.DMA,
        ],
    )
    def k(idx_hbm, table_hbm, out_hbm, idx_v, buf0, buf1, sem0, sem1):
        wid = lax.axis_index("s") * _NC + lax.axis_index("c")
        base = wid * n_per
        pltpu.sync_copy(idx_hbm.at[wid], idx_v)
        bufs = (buf0, buf1)
        sems = (sem0, sem1)

        def gather(j):
            b = j % 2
            return pltpu.make_async_copy(
                table_hbm.at[idx_v.at[pl.ds(offs[j], sizes[j])]],
                bufs[b].at[pl.ds(0, sizes[j])],
                sems[b],
            )

        gather(0).start()
        for j in range(n):
            if j + 1 < n:
                gather(j + 1).start()
            gather(j).wait()
            pltpu.sync_copy(
                bufs[j % 2].at[pl.ds(0, sizes[j])],
                out_hbm.at[pl.ds(base + offs[j], sizes[j])],
            )

    return k(idx, W_E)


def kernel(tokens, W_E):
    B, T = tokens.shape
    V, D = W_E.shape
    idx = tokens.reshape(_NW, (B * T) // _NW).astype(jnp.int32)
    out = _sc_embed(idx, W_E, D)
    return out.reshape(B, T, D)
